# Initial kernel scaffold; baseline (speedup 1.0000x reference)
import jax
import jax.numpy as jnp
from jax.experimental import pallas as pl


def kernel(node_features, edge_index, weight0, bias0, weight1, bias1, hidden_dim):
    raise NotImplementedError("write your pallas kernel here")


# speedup vs baseline: 4.9741x; 4.9741x over previous
import functools

import jax
import jax.numpy as jnp
from jax import lax
from jax.experimental import pallas as pl
from jax.experimental.pallas import tpu as pltpu
from jax.experimental.pallas import tpu_sc as plsc

_BR = 1000


def _dense(x, wt, b):
    N, D = x.shape

    def body(x_ref, w_ref, b_ref, o_ref):
        o_ref[...] = (
            jnp.dot(x_ref[...], w_ref[...], preferred_element_type=jnp.float32)
            + b_ref[...]
        )

    return pl.pallas_call(
        body,
        grid=(N // _BR,),
        in_specs=[
            pl.BlockSpec((_BR, D), lambda i: (i, 0)),
            pl.BlockSpec((D, D), lambda i: (0, 0)),
            pl.BlockSpec((1, D), lambda i: (0, 0)),
        ],
        out_specs=pl.BlockSpec((_BR, D), lambda i: (i, 0)),
        out_shape=jax.ShapeDtypeStruct((N, D), jnp.float32),
    )(x, wt, b)


def _combine_relu_dense(t, p, wt, b):
    N, D = t.shape
    NC = p.shape[0]

    def body(t_ref, p_ref, w_ref, b_ref, o_ref):
        h = t_ref[...] + jnp.sum(p_ref[...], axis=0)
        h = jnp.maximum(h, 0.0)
        o_ref[...] = (
            jnp.dot(h, w_ref[...], preferred_element_type=jnp.float32) + b_ref[...]
        )

    return pl.pallas_call(
        body,
        grid=(N // _BR,),
        in_specs=[
            pl.BlockSpec((_BR, D), lambda i: (i, 0)),
            pl.BlockSpec((NC, _BR, D), lambda i: (0, i, 0)),
            pl.BlockSpec((D, D), lambda i: (0, 0)),
            pl.BlockSpec((1, D), lambda i: (0, 0)),
        ],
        out_specs=pl.BlockSpec((_BR, D), lambda i: (i, 0)),
        out_shape=jax.ShapeDtypeStruct((N, D), jnp.float32),
    )(t, p, wt, b)


def _combine(t, p):
    N, D = t.shape
    NC = p.shape[0]

    def body(t_ref, p_ref, o_ref):
        o_ref[...] = t_ref[...] + jnp.sum(p_ref[...], axis=0)

    return pl.pallas_call(
        body,
        grid=(N // _BR,),
        in_specs=[
            pl.BlockSpec((_BR, D), lambda i: (i, 0)),
            pl.BlockSpec((NC, _BR, D), lambda i: (0, i, 0)),
        ],
        out_specs=pl.BlockSpec((_BR, D), lambda i: (i, 0)),
        out_shape=jax.ShapeDtypeStruct((N, D), jnp.float32),
    )(t, p)


def _sc_aggregate(t, row, col):
    N, D = t.shape
    E = row.shape[0]
    info = plsc.get_sparse_core_info()
    NC, NS = info.num_cores, info.num_subcores
    NW = NC * NS
    assert E % NW == 0 and N % NS == 0 and D % 16 == 0
    EPW = E // NW
    CH = max(c for c in range(8, 129, 8) if EPW % c == 0)
    RPT = N // NS
    ZR = max(z for z in range(1, 129) if RPT % z == 0)
    mesh = plsc.VectorSubcoreMesh(core_axis_name="c", subcore_axis_name="s")

    @functools.partial(
        pl.kernel,
        out_type=jax.ShapeDtypeStruct((NC, NS, RPT, D), jnp.float32),
        mesh=mesh,
        scratch_types=[
            pltpu.VMEM((CH,), jnp.int32),
            pltpu.VMEM((CH,), jnp.int32),
            pltpu.VMEM((CH, D), jnp.float32),
            pltpu.VMEM((ZR, D), jnp.float32),
            pltpu.VMEM_SHARED((N, D), jnp.float32),
            pltpu.SemaphoreType.DMA,
        ],
    )
    def k(t_hbm, row_hbm, col_hbm, out_hbm, rowv, colv, rows, zbuf, acc, sem):
        cid = lax.axis_index("c")
        sid = lax.axis_index("s")
        wid = sid * NC + cid

        nsl = D // 16

        def zb(i, c):
            zbuf[i // nsl, pl.ds((i % nsl) * 16, 16)] = jnp.zeros((16,), jnp.float32)
            return c

        lax.fori_loop(0, ZR * nsl, zb, 0)

        def zc(i, c):
            pltpu.sync_copy(zbuf, acc.at[pl.ds(sid * RPT + i * ZR, ZR)])
            return c

        lax.fori_loop(0, RPT // ZR, zc, 0)
        plsc.subcore_barrier()

        base0 = wid * EPW

        def body(i, c):
            base = base0 + i * CH
            pltpu.sync_copy(row_hbm.at[pl.ds(base, CH)], rowv)
            pltpu.sync_copy(col_hbm.at[pl.ds(base, CH)], colv)
            pltpu.async_copy(t_hbm.at[rowv], rows, sem).wait()
            pltpu.sync_copy(rows, acc.at[colv], add=True)
            return c

        lax.fori_loop(0, EPW // CH, body, 0)
        plsc.subcore_barrier()
        pltpu.sync_copy(acc.at[pl.ds(sid * RPT, RPT)], out_hbm.at[cid, sid])

    return k(t, row, col).reshape(NC, N, D)


def kernel(node_features, edge_index, weight0, bias0, weight1, bias1, hidden_dim):
    del hidden_dim
    row = edge_index[0]
    col = edge_index[1]
    wt0 = jnp.transpose(weight0[0]) * 0.5
    b0 = bias0 * 0.5
    wt1 = jnp.transpose(weight1[0]) * 0.5
    b1 = bias1 * 0.5
    t0 = _dense(node_features, wt0, b0)
    p0 = _sc_aggregate(t0, row, col)
    t1 = _combine_relu_dense(t0, p0, wt1, b1)
    p1 = _sc_aggregate(t1, row, col)
    return _combine(t1, p1)
